# Initial kernel scaffold; baseline (speedup 1.0000x reference)
import jax
import jax.numpy as jnp
from jax.experimental import pallas as pl


def kernel(input_q, input_k, input_v, rpe_knn_embeddings, knn_idx, Wq, bq, Wq1, bq1, Wk, bk, Wv, bv, Wp, bp):
    raise NotImplementedError("write your pallas kernel here")


# speedup vs baseline: 2.2015x; 2.2015x over previous
import jax
import jax.numpy as jnp
import numpy as np
from jax.experimental import pallas as pl
from jax.experimental.pallas import tpu as pltpu

_B, _N, _C, _H, _K = 1, 2048, 768, 12, 35
_D = _C // _H
_SCALE = 1.0 / np.sqrt(_D)

_PROJ_NB = 256
_ATTN_NB = 64


def _proj_body(xq_ref, xk_ref, xv_ref, wq_ref, bq_ref, wq1_ref, bq1_ref,
               wk_ref, bk_ref, wv_ref, bv_ref,
               q_ref, q1_ref, k_ref, v_ref):
    xq = xq_ref[...]
    q_ref[...] = jnp.dot(xq, wq_ref[...], preferred_element_type=jnp.float32) + bq_ref[...]
    q1_ref[...] = jnp.dot(xq, wq1_ref[...], preferred_element_type=jnp.float32) + bq1_ref[...]
    k_ref[...] = jnp.dot(xk_ref[...], wk_ref[...], preferred_element_type=jnp.float32) + bk_ref[...]
    v_ref[...] = jnp.dot(xv_ref[...], wv_ref[...], preferred_element_type=jnp.float32) + bv_ref[...]


def _attn_body(q_ref, q1_ref, kall_ref, vall_ref, idx_ref, rpe_ref,
               wp_ref, bp_ref, pool_ref, poolt_ref,
               hid_ref, probs_ref, vnb_scr):
    nb = q_ref.shape[0]
    q = q_ref[...]
    q1 = q1_ref[...]
    kall = kall_ref[...]
    vall = vall_ref[...]
    wp = wp_ref[...]
    bp = bp_ref[...]
    pool = pool_ref[...]
    iota = jax.lax.broadcasted_iota(jnp.int32, (nb, _N), 1)

    for k in range(_K):
        idx_k = idx_ref[:, k][:, None]
        oh = (iota == idx_k).astype(jnp.float32)
        knb = jnp.dot(oh, kall, preferred_element_type=jnp.float32)
        vnb = jnp.dot(oh, vall, preferred_element_type=jnp.float32)
        vnb_scr[k] = vnb
        p_k = jnp.dot(rpe_ref[:, k, :], wp, preferred_element_type=jnp.float32) + bp
        e = jnp.dot(q * knb, pool, preferred_element_type=jnp.float32)
        ep = jnp.dot(q1 * p_k, pool, preferred_element_type=jnp.float32)
        probs_ref[:, k, :] = (e + ep) * _SCALE

    s = probs_ref[...]
    m = jnp.max(s, axis=1, keepdims=True)
    ex = jnp.exp(s - m)
    pr = ex / jnp.sum(ex, axis=1, keepdims=True)
    probs_ref[...] = pr

    acc = jnp.zeros((nb, _C), dtype=jnp.float32)
    poolt = poolt_ref[...]
    for k in range(_K):
        w = jnp.dot(pr[:, k, :], poolt, preferred_element_type=jnp.float32)
        acc = acc + w * vnb_scr[k]
    hid_ref[...] = acc


def kernel(input_q, input_k, input_v, rpe_knn_embeddings, knn_idx,
           Wq, bq, Wq1, bq1, Wk, bk, Wv, bv, Wp, bp):
    xq = input_q.reshape(_N, _C)
    xk = input_k.reshape(_N, _C)
    xv = input_v.reshape(_N, _C)
    rpe = rpe_knn_embeddings.reshape(_N, _K, _C)
    idx = knn_idx.reshape(_N, _K).astype(jnp.int32)

    b2 = lambda b: b.reshape(1, _C)

    q, q1, k_all, v_all = pl.pallas_call(
        _proj_body,
        grid=(_N // _PROJ_NB,),
        in_specs=[
            pl.BlockSpec((_PROJ_NB, _C), lambda i: (i, 0)),
            pl.BlockSpec((_PROJ_NB, _C), lambda i: (i, 0)),
            pl.BlockSpec((_PROJ_NB, _C), lambda i: (i, 0)),
        ] + [
            spec for _ in range(4) for spec in (
                pl.BlockSpec((_C, _C), lambda i: (0, 0)),
                pl.BlockSpec((1, _C), lambda i: (0, 0)),
            )
        ],
        out_specs=[pl.BlockSpec((_PROJ_NB, _C), lambda i: (i, 0))] * 4,
        out_shape=[jax.ShapeDtypeStruct((_N, _C), jnp.float32)] * 4,
    )(xq, xk, xv, Wq.T, b2(bq), Wq1.T, b2(bq1), Wk.T, b2(bk), Wv.T, b2(bv))

    pool = jnp.repeat(jnp.eye(_H, dtype=jnp.float32), _D, axis=0)

    hid, probs_raw = pl.pallas_call(
        _attn_body,
        grid=(_N // _ATTN_NB,),
        in_specs=[
            pl.BlockSpec((_ATTN_NB, _C), lambda i: (i, 0)),
            pl.BlockSpec((_ATTN_NB, _C), lambda i: (i, 0)),
            pl.BlockSpec((_N, _C), lambda i: (0, 0)),
            pl.BlockSpec((_N, _C), lambda i: (0, 0)),
            pl.BlockSpec((_ATTN_NB, _K), lambda i: (i, 0)),
            pl.BlockSpec((_ATTN_NB, _K, _C), lambda i: (i, 0, 0)),
            pl.BlockSpec((_C, _C), lambda i: (0, 0)),
            pl.BlockSpec((1, _C), lambda i: (0, 0)),
            pl.BlockSpec((_C, _H), lambda i: (0, 0)),
            pl.BlockSpec((_H, _C), lambda i: (0, 0)),
        ],
        out_specs=[
            pl.BlockSpec((_ATTN_NB, _C), lambda i: (i, 0)),
            pl.BlockSpec((_ATTN_NB, _K, _H), lambda i: (i, 0, 0)),
        ],
        out_shape=[
            jax.ShapeDtypeStruct((_N, _C), jnp.float32),
            jax.ShapeDtypeStruct((_N, _K, _H), jnp.float32),
        ],
        scratch_shapes=[pltpu.VMEM((_K, _ATTN_NB, _C), jnp.float32)],
    )(q, q1, k_all, v_all, idx, rpe, Wp.T, b2(bp), pool, pool.T)

    hidden = hid.reshape(_B, _N, _C)
    attention_probs = probs_raw.transpose(0, 2, 1).reshape(_B, _N, _H, _K)
    return (hidden, attention_probs)


# speedup vs baseline: 3.3248x; 1.5102x over previous
import jax
import jax.numpy as jnp
import numpy as np
from jax.experimental import pallas as pl
from jax.experimental.pallas import tpu as pltpu

_B, _N, _C, _H, _K = 1, 2048, 768, 12, 35
_D = _C // _H
_SCALE = 1.0 / np.sqrt(_D)

_PROJ_NB = 256
_ATTN_NB = 128


def _proj_body(xq_ref, xk_ref, xv_ref, wq_ref, bq_ref, wq1_ref, bq1_ref,
               wk_ref, bk_ref, wv_ref, bv_ref,
               q_ref, q1_ref, k_ref, v_ref, kbf_ref, vbf_ref):
    xq = xq_ref[...]
    q_ref[...] = jnp.dot(xq, wq_ref[...], preferred_element_type=jnp.float32) + bq_ref[...]
    q1_ref[...] = jnp.dot(xq, wq1_ref[...], preferred_element_type=jnp.float32) + bq1_ref[...]
    k = jnp.dot(xk_ref[...], wk_ref[...], preferred_element_type=jnp.float32) + bk_ref[...]
    v = jnp.dot(xv_ref[...], wv_ref[...], preferred_element_type=jnp.float32) + bv_ref[...]
    k_ref[...] = k
    v_ref[...] = v
    kbf_ref[...] = k.astype(jnp.bfloat16)
    vbf_ref[...] = v.astype(jnp.bfloat16)


def _attn_body(q_ref, q1_ref, kall_ref, vall_ref, idx_ref, rpe_ref,
               wp_ref, bp_ref, pool_ref, poolt_ref,
               hid_ref, probs_ref, vnb_scr):
    nb = q_ref.shape[0]
    q = q_ref[...]
    q1 = q1_ref[...]
    kall = kall_ref[...]
    vall = vall_ref[...]
    wp = wp_ref[...]
    bp = bp_ref[...]
    pool = pool_ref[...]
    iota = jax.lax.broadcasted_iota(jnp.int32, (nb, _N), 1)

    for k in range(_K):
        idx_k = idx_ref[:, k][:, None]
        oh = (iota == idx_k).astype(jnp.bfloat16)
        knb = jnp.dot(oh, kall, preferred_element_type=jnp.float32)
        vnb = jnp.dot(oh, vall, preferred_element_type=jnp.float32)
        vnb_scr[k] = vnb.astype(jnp.bfloat16)
        rpe_k = rpe_ref[:, k, :].astype(jnp.bfloat16)
        p_k = jnp.dot(rpe_k, wp, preferred_element_type=jnp.float32) + bp
        e = jnp.dot(q * knb, pool, preferred_element_type=jnp.float32)
        ep = jnp.dot(q1 * p_k, pool, preferred_element_type=jnp.float32)
        probs_ref[:, k, :] = (e + ep) * _SCALE

    s = probs_ref[...]
    m = jnp.max(s, axis=1, keepdims=True)
    ex = jnp.exp(s - m)
    pr = ex / jnp.sum(ex, axis=1, keepdims=True)
    probs_ref[...] = pr

    acc = jnp.zeros((nb, _C), dtype=jnp.float32)
    poolt = poolt_ref[...]
    for k in range(_K):
        w = jnp.dot(pr[:, k, :], poolt, preferred_element_type=jnp.float32)
        acc = acc + w * vnb_scr[k].astype(jnp.float32)
    hid_ref[...] = acc


def kernel(input_q, input_k, input_v, rpe_knn_embeddings, knn_idx,
           Wq, bq, Wq1, bq1, Wk, bk, Wv, bv, Wp, bp):
    xq = input_q.reshape(_N, _C)
    xk = input_k.reshape(_N, _C)
    xv = input_v.reshape(_N, _C)
    rpe = rpe_knn_embeddings.reshape(_N, _K, _C)
    idx = knn_idx.reshape(_N, _K).astype(jnp.int32)

    b2 = lambda b: b.reshape(1, _C)

    q, q1, k_all, v_all, k_bf, v_bf = pl.pallas_call(
        _proj_body,
        grid=(_N // _PROJ_NB,),
        in_specs=[
            pl.BlockSpec((_PROJ_NB, _C), lambda i: (i, 0)),
            pl.BlockSpec((_PROJ_NB, _C), lambda i: (i, 0)),
            pl.BlockSpec((_PROJ_NB, _C), lambda i: (i, 0)),
        ] + [
            spec for _ in range(4) for spec in (
                pl.BlockSpec((_C, _C), lambda i: (0, 0)),
                pl.BlockSpec((1, _C), lambda i: (0, 0)),
            )
        ],
        out_specs=[pl.BlockSpec((_PROJ_NB, _C), lambda i: (i, 0))] * 6,
        out_shape=[jax.ShapeDtypeStruct((_N, _C), jnp.float32)] * 4
        + [jax.ShapeDtypeStruct((_N, _C), jnp.bfloat16)] * 2,
    )(xq, xk, xv, Wq.T, b2(bq), Wq1.T, b2(bq1), Wk.T, b2(bk), Wv.T, b2(bv))

    pool = jnp.repeat(jnp.eye(_H, dtype=jnp.float32), _D, axis=0)

    hid, probs_raw = pl.pallas_call(
        _attn_body,
        grid=(_N // _ATTN_NB,),
        in_specs=[
            pl.BlockSpec((_ATTN_NB, _C), lambda i: (i, 0)),
            pl.BlockSpec((_ATTN_NB, _C), lambda i: (i, 0)),
            pl.BlockSpec((_N, _C), lambda i: (0, 0)),
            pl.BlockSpec((_N, _C), lambda i: (0, 0)),
            pl.BlockSpec((_ATTN_NB, _K), lambda i: (i, 0)),
            pl.BlockSpec((_ATTN_NB, _K, _C), lambda i: (i, 0, 0)),
            pl.BlockSpec((_C, _C), lambda i: (0, 0)),
            pl.BlockSpec((1, _C), lambda i: (0, 0)),
            pl.BlockSpec((_C, _H), lambda i: (0, 0)),
            pl.BlockSpec((_H, _C), lambda i: (0, 0)),
        ],
        out_specs=[
            pl.BlockSpec((_ATTN_NB, _C), lambda i: (i, 0)),
            pl.BlockSpec((_ATTN_NB, _K, _H), lambda i: (i, 0, 0)),
        ],
        out_shape=[
            jax.ShapeDtypeStruct((_N, _C), jnp.float32),
            jax.ShapeDtypeStruct((_N, _K, _H), jnp.float32),
        ],
        scratch_shapes=[pltpu.VMEM((_K, _ATTN_NB, _C), jnp.bfloat16)],
    )(q, q1, k_bf, v_bf, idx, rpe, Wp.T.astype(jnp.bfloat16), b2(bp), pool, pool.T)

    hidden = hid.reshape(_B, _N, _C)
    attention_probs = probs_raw.transpose(0, 2, 1).reshape(_B, _N, _H, _K)
    return (hidden, attention_probs)
